# baseline (speedup 1.0000x reference)
import functools

import jax
import jax.numpy as jnp
from jax import lax
from jax.experimental import pallas as pl
from jax.experimental.pallas import tpu as pltpu
from jax.experimental.pallas import tpu_sc as plsc

S, N = 2048, 4
D = 1024
TOKENS = S * N
L = 16
DJ = D // L
EPS = 1e-12

_info = plsc.get_sparse_core_info()
NC, NS = _info.num_cores, _info.num_subcores
NW = NC * NS
PER_W = TOKENS // NW
C = 32
NCHUNK = PER_W // C


_GATHER_DN = lax.GatherDimensionNumbers(
    offset_dims=(), collapsed_slice_dims=(0,), start_index_map=(0,))


def _bcast_lane(vec, lane):
    idx = jnp.full((L, 1), lane, dtype=jnp.int32)
    return lax.gather(vec, idx, _GATHER_DN, (1,),
                      mode=lax.GatherScatterMode.PROMISE_IN_BOUNDS)


def _shuffle(vec, idx):
    return lax.gather(vec, idx[:, None], _GATHER_DN, (1,),
                      mode=lax.GatherScatterMode.PROMISE_IN_BOUNDS)


def _allreduce_sum(vec):
    lanes = lax.iota(jnp.int32, L)
    for k in (1, 2, 4, 8):
        vec = vec + _shuffle(vec, lax.bitwise_xor(lanes, k))
    return vec


def _rsqrt(x):
    i = lax.bitcast_convert_type(x, jnp.int32)
    y = lax.bitcast_convert_type(
        jnp.int32(0x5F3759DF) - lax.shift_right_arithmetic(i, 1), jnp.float32)
    for _ in range(3):
        y = y * (1.5 - 0.5 * x * y * y)
    return y


def _sc_kernel(tok_hbm, posid_hbm, typef_hbm, word_hbm, pos_hbm, type_hbm,
               gamma_hbm, beta_hbm, out_hbm,
               tokbuf, posbuf, typbuf, ttbuf, difbuf, gbuf, bbuf, wbuf, pbuf):
    wid = lax.axis_index("s") * NC + lax.axis_index("c")
    base = wid * PER_W

    pltpu.sync_copy(tok_hbm.at[pl.ds(base, PER_W)], tokbuf)
    pltpu.sync_copy(posid_hbm.at[pl.ds(base, PER_W)], posbuf)
    pltpu.sync_copy(typef_hbm.at[pl.ds(base, PER_W)], typbuf)
    pltpu.sync_copy(type_hbm, ttbuf)
    pltpu.sync_copy(gamma_hbm, gbuf)
    pltpu.sync_copy(beta_hbm, bbuf)
    for j in range(DJ):
        sl = pl.ds(j * L, L)
        difbuf[sl] = ttbuf[1, sl] - ttbuf[0, sl]

    def chunk_body(c, carry):
        pltpu.sync_copy(word_hbm.at[tokbuf.at[pl.ds(c * C, C)]], wbuf)
        pltpu.sync_copy(pos_hbm.at[posbuf.at[pl.ds(c * C, C)]], pbuf)

        def grp_body(g, carry2):
            tvec = typbuf[pl.ds(c * C + g * L, L)]

            def tok_body(lane, carry3):
                row = g * L + lane
                tf = _bcast_lane(tvec, lane)
                s = jnp.zeros((L,), jnp.float32)
                ss = jnp.zeros((L,), jnp.float32)
                for j in range(DJ):
                    sl = pl.ds(j * L, L)
                    a = (wbuf[row, sl] + pbuf[row, sl]
                         + ttbuf[0, sl] + tf * difbuf[sl])
                    wbuf[row, sl] = a
                    s = s + a
                    ss = ss + a * a
                tot = _allreduce_sum(s)
                tots = _allreduce_sum(ss)
                mean = tot * (1.0 / D)
                var = tots * (1.0 / D) - mean * mean
                inv = _rsqrt(var + EPS)
                for j in range(DJ):
                    sl = pl.ds(j * L, L)
                    wbuf[row, sl] = ((wbuf[row, sl] - mean) * inv * gbuf[sl]
                                     + bbuf[sl])
                return carry3

            return lax.fori_loop(0, L, tok_body, carry2)

        lax.fori_loop(0, C // L, grp_body, 0)
        pltpu.sync_copy(wbuf, out_hbm.at[pl.ds(base + c * C, C)])
        return carry

    lax.fori_loop(0, NCHUNK, chunk_body, 0)


def kernel(token_ids, type_ids, position_ids, word_table, type_table,
           pos_table, gamma, beta):
    tok = token_ids.reshape(-1).astype(jnp.int32)
    posid = position_ids.reshape(-1).astype(jnp.int32)
    typef = type_ids.reshape(-1).astype(jnp.float32)

    mesh = plsc.VectorSubcoreMesh(core_axis_name="c", subcore_axis_name="s")
    f = functools.partial(
        pl.kernel,
        mesh=mesh,
        out_type=jax.ShapeDtypeStruct((TOKENS, D), jnp.float32),
        scratch_types=[
            pltpu.VMEM((PER_W,), jnp.int32),
            pltpu.VMEM((PER_W,), jnp.int32),
            pltpu.VMEM((PER_W,), jnp.float32),
            pltpu.VMEM((2, D), jnp.float32),
            pltpu.VMEM((D,), jnp.float32),
            pltpu.VMEM((D,), jnp.float32),
            pltpu.VMEM((D,), jnp.float32),
            pltpu.VMEM((C, D), jnp.float32),
            pltpu.VMEM((C, D), jnp.float32),
        ],
    )(_sc_kernel)
    out = f(tok, posid, typef, word_table, pos_table, type_table, gamma, beta)
    return out.reshape(S, N, D)
